# baseline (speedup 1.0000x reference)
import jax
import jax.numpy as jnp
from jax.experimental import pallas as pl
from jax.experimental.pallas import tpu as pltpu

N = 10000
H = 64
O = 3


def _gat_conv(x, src, dst, W, a_src, a_dst, b):
    n = x.shape[0]
    h = x @ W
    alpha_src = (h * a_src).sum(-1)
    alpha_dst = (h * a_dst).sum(-1)
    e = jax.nn.leaky_relu(alpha_src[src] + alpha_dst[dst], negative_slope=0.2)
    m = jax.ops.segment_max(e, dst, num_segments=n)
    p = jnp.exp(e - m[dst])
    denom = jax.ops.segment_sum(p, dst, num_segments=n)
    alpha = p / (denom[dst] + 1e-16)
    out = jax.ops.segment_sum(h[src] * alpha[:, None], dst, num_segments=n)
    return out + b


def _head_kernel(h_ref, gamma_ref, beta_ref, w1_ref, b1_ref, w2_ref, b2_ref,
                 out_ref):
    h = h_ref[...]
    mean = jnp.mean(h, axis=0, keepdims=True)
    var = jnp.mean((h - mean) ** 2, axis=0, keepdims=True)
    hn = (h - mean) * jax.lax.rsqrt(var + 1e-5) * gamma_ref[...] + beta_ref[...]
    z = jnp.maximum(jnp.dot(hn, w1_ref[...], preferred_element_type=jnp.float32)
                    + b1_ref[...], 0.0)
    out_ref[...] = (jnp.dot(z, w2_ref[...], preferred_element_type=jnp.float32)
                    + b2_ref[...])


def _head(h, bn_gamma, bn_beta, lin1_W, lin1_b, lin2_W, lin2_b):
    w2p = jnp.zeros((H, 128), jnp.float32).at[:, :O].set(lin2_W)
    b2p = jnp.zeros((128,), jnp.float32).at[:O].set(lin2_b)
    out = pl.pallas_call(
        _head_kernel,
        out_shape=jax.ShapeDtypeStruct((N, 128), jnp.float32),
    )(h, bn_gamma[None, :], bn_beta[None, :], lin1_W, lin1_b[None, :],
      w2p, b2p[None, :])
    return out[:, :O]


def kernel(x, edge_index, W1, a_src1, a_dst1, b1, W2, a_src2, a_dst2, b2,
           W3, a_src3, a_dst3, b3, bn_gamma, bn_beta,
           lin1_W, lin1_b, lin2_W, lin2_b):
    n = x.shape[0]
    loops = jnp.arange(n, dtype=edge_index.dtype)
    src = jnp.concatenate([edge_index[0], loops])
    dst = jnp.concatenate([edge_index[1], loops])
    h = jax.nn.relu(_gat_conv(x, src, dst, W1, a_src1, a_dst1, b1))
    h = jax.nn.relu(_gat_conv(h, src, dst, W2, a_src2, a_dst2, b2))
    h = jax.nn.relu(_gat_conv(h, src, dst, W3, a_src3, a_dst3, b3))
    return _head(h, bn_gamma, bn_beta, lin1_W, lin1_b, lin2_W, lin2_b)


# speedup vs baseline: 8.5559x; 8.5559x over previous
import functools

import jax
import jax.numpy as jnp
from jax import lax
from jax.experimental import pallas as pl
from jax.experimental.pallas import tpu as pltpu
from jax.experimental.pallas import tpu_sc as plsc

N = 10000
D = 128
H = 64
O = 3

NP = 10240
NPS = NP // 16
C = 512
RPC = C // 128
NW = 32


def _ceil_to(x, m):
    return (x + m - 1) // m * m


def _edge_a_body(ncH, T, src2, dst2, asrc, adst, mub, p_out, den_out,
                 asrc_v, adst_v, mub_v, sidx2_v, didx2_v, p_v, zb_v,
                 den_sh):
    c = lax.axis_index("c")
    s = lax.axis_index("s")
    w = s * 2 + c
    pltpu.sync_copy(asrc, asrc_v)
    pltpu.sync_copy(adst, adst_v)
    pltpu.sync_copy(mub, mub_v)
    for z in range(NPS // 16):
        zb_v[pl.ds(z * 16, 16)] = jnp.zeros((16,), jnp.float32)
    pltpu.sync_copy(zb_v, den_sh.at[pl.ds(s * NPS, NPS)])
    plsc.subcore_barrier()

    def chunk(i, carry):
        base = w * T + i * C
        row0 = w * (T // 128) + i * RPC
        pltpu.sync_copy(src2.at[pl.ds(row0, RPC)], sidx2_v)
        pltpu.sync_copy(dst2.at[pl.ds(row0, RPC)], didx2_v)
        for k in range(C // 16):
            r, kk = divmod(k, 8)
            si = sidx2_v[r, pl.ds(kk * 16, 16)]
            di = didx2_v[r, pl.ds(kk * 16, 16)]
            a1 = plsc.load_gather(asrc_v, [si])
            a2 = plsc.load_gather(adst_v, [di])
            mu = plsc.load_gather(mub_v, [di])
            x = a1 + a2
            e = jnp.maximum(x, 0.2 * x)
            p_v[pl.ds(k * 16, 16)] = jnp.exp(e - mu)
        pltpu.sync_copy(p_v, p_out.at[pl.ds(base, C)])
        for r in range(RPC):
            pltpu.sync_copy(p_v.at[pl.ds(r * 128, 128)],
                            den_sh.at[didx2_v.at[r]], add=True)
        return carry

    lax.fori_loop(0, ncH, chunk, 0)
    plsc.subcore_barrier()
    pltpu.sync_copy(den_sh.at[pl.ds(s * NPS, NPS)],
                    den_out.at[c, pl.ds(s * NPS, NPS)])


def _edge_b_body(ncH, T, src2, dst2, p_in, h, dpart, out_part,
                 invd_v, dtmp_v, sidx2_v, didx2_v, p_v, hbuf, zb2_v,
                 out_sh):
    c = lax.axis_index("c")
    s = lax.axis_index("s")
    w = s * 2 + c
    pltpu.sync_copy(dpart.at[0], invd_v)
    pltpu.sync_copy(dpart.at[1], dtmp_v)

    def inv_body(k, carry):
        v = invd_v[pl.ds(k * 16, 16)] + dtmp_v[pl.ds(k * 16, 16)]
        invd_v[pl.ds(k * 16, 16)] = 1.0 / (v + 1e-16)
        return carry

    lax.fori_loop(0, NP // 16, inv_body, 0)

    def zb_body(r, carry):
        for q in range(4):
            zb2_v[r, pl.ds(q * 16, 16)] = jnp.zeros((16,), jnp.float32)
        return carry

    lax.fori_loop(0, 128, zb_body, 0)
    for bloc in range(NPS // 128):
        pltpu.sync_copy(zb2_v, out_sh.at[pl.ds(s * NPS + bloc * 128, 128)])
    plsc.subcore_barrier()

    iota16 = lax.iota(jnp.int32, 16)

    def chunk(i, carry):
        base = w * T + i * C
        row0 = w * (T // 128) + i * RPC
        pltpu.sync_copy(src2.at[pl.ds(row0, RPC)], sidx2_v)
        pltpu.sync_copy(dst2.at[pl.ds(row0, RPC)], didx2_v)
        pltpu.sync_copy(p_in.at[pl.ds(base, C)], p_v)
        for r in range(RPC):
            pltpu.sync_copy(h.at[sidx2_v.at[r]], hbuf.at[pl.ds(r * 128, 128)])

        for r in range(RPC):
            def scale(kk, carry2, r=r):
                di = didx2_v[r, pl.ds(kk * 16, 16)]
                iv = plsc.load_gather(invd_v, [di])
                al = p_v[pl.ds(r * 128 + kk * 16, 16)] * iv
                rows = r * 128 + kk * 16 + iota16
                for j in range(H):
                    cols = jnp.full((16,), j, jnp.int32)
                    v = plsc.load_gather(hbuf, [rows, cols])
                    plsc.store_scatter(hbuf, [rows, cols], v * al)
                return carry2

            lax.fori_loop(0, 8, scale, 0)
        for r in range(RPC):
            pltpu.sync_copy(hbuf.at[pl.ds(r * 128, 128)],
                            out_sh.at[didx2_v.at[r]], add=True)
        return carry

    lax.fori_loop(0, ncH, chunk, 0)
    plsc.subcore_barrier()
    pltpu.sync_copy(out_sh.at[pl.ds(s * NPS, NPS)],
                    out_part.at[c, pl.ds(s * NPS, NPS)])


def _make_edge_kernels(EAP):
    T = EAP // NW
    ncH = T // C
    mesh = plsc.VectorSubcoreMesh(core_axis_name="c", subcore_axis_name="s")
    f32 = jnp.float32
    i32 = jnp.int32
    cparams = pltpu.CompilerParams(needs_layout_passes=False,
                                   use_tc_tiling_on_sc=False)
    edge_a = functools.partial(
        pl.kernel,
        functools.partial(_edge_a_body, ncH, T),
        out_type=(jax.ShapeDtypeStruct((EAP,), f32),
                  jax.ShapeDtypeStruct((2, NP), f32)),
        mesh=mesh,
        scratch_types=[
            pltpu.VMEM((NP,), f32), pltpu.VMEM((NP,), f32),
            pltpu.VMEM((NP,), f32),
            pltpu.VMEM((RPC, 128), i32), pltpu.VMEM((RPC, 128), i32),
            pltpu.VMEM((C,), f32), pltpu.VMEM((NPS,), f32),
            pltpu.VMEM_SHARED((NP,), f32),
        ],
        compiler_params=cparams,
    )()
    edge_b = functools.partial(
        pl.kernel,
        functools.partial(_edge_b_body, ncH, T),
        out_type=jax.ShapeDtypeStruct((2, NP, H), f32),
        mesh=mesh,
        scratch_types=[
            pltpu.VMEM((NP,), f32), pltpu.VMEM((NP,), f32),
            pltpu.VMEM((RPC, 128), i32), pltpu.VMEM((RPC, 128), i32),
            pltpu.VMEM((C,), f32), pltpu.VMEM((C, H), f32),
            pltpu.VMEM((128, H), f32),
            pltpu.VMEM_SHARED((NP, H), f32),
        ],
        compiler_params=cparams,
    )()
    return edge_a, edge_b


def _prep1_body(x_ref, w_ref, asv_ref, adv_ref, oh, oas, oad, omu):
    h = jnp.dot(x_ref[...], w_ref[...], preferred_element_type=jnp.float32)
    oh[...] = h
    a_s = jnp.sum(h * asv_ref[...], axis=1, keepdims=True)
    a_d = jnp.sum(h * adv_ref[...], axis=1, keepdims=True)
    oas[...] = a_s
    oad[...] = a_d
    z = jnp.max(a_s) + a_d
    omu[...] = jnp.maximum(z, 0.2 * z)


def _prep23_body(part_ref, b_ref, w_ref, asv_ref, adv_ref, oh, oas, oad, omu):
    g = jnp.maximum(part_ref[0, :N, :] + part_ref[1, :N, :] + b_ref[...], 0.0)
    h = jnp.dot(g, w_ref[...], preferred_element_type=jnp.float32)
    oh[...] = h
    a_s = jnp.sum(h * asv_ref[...], axis=1, keepdims=True)
    a_d = jnp.sum(h * adv_ref[...], axis=1, keepdims=True)
    oas[...] = a_s
    oad[...] = a_d
    z = jnp.max(a_s) + a_d
    omu[...] = jnp.maximum(z, 0.2 * z)


def _head_body(part_ref, b_ref, gamma_ref, beta_ref, w1_ref, b1_ref, w2_ref,
               b2_ref, out_ref):
    g = jnp.maximum(part_ref[0, :N, :] + part_ref[1, :N, :] + b_ref[...], 0.0)
    mean = jnp.mean(g, axis=0, keepdims=True)
    var = jnp.mean((g - mean) ** 2, axis=0, keepdims=True)
    hn = (g - mean) * lax.rsqrt(var + 1e-5) * gamma_ref[...] + beta_ref[...]
    z = jnp.maximum(jnp.dot(hn, w1_ref[...], preferred_element_type=jnp.float32)
                    + b1_ref[...], 0.0)
    out_ref[...] = (jnp.dot(z, w2_ref[...], preferred_element_type=jnp.float32)
                    + b2_ref[...])


_f32 = jnp.float32

_prep1 = pl.pallas_call(
    _prep1_body,
    out_shape=(jax.ShapeDtypeStruct((N, H), _f32),
               jax.ShapeDtypeStruct((N, 1), _f32),
               jax.ShapeDtypeStruct((N, 1), _f32),
               jax.ShapeDtypeStruct((N, 1), _f32)))

_prep23 = pl.pallas_call(
    _prep23_body,
    out_shape=(jax.ShapeDtypeStruct((N, H), _f32),
               jax.ShapeDtypeStruct((N, 1), _f32),
               jax.ShapeDtypeStruct((N, 1), _f32),
               jax.ShapeDtypeStruct((N, 1), _f32)))

_head = pl.pallas_call(
    _head_body,
    out_shape=jax.ShapeDtypeStruct((N, 128), _f32))


def _pad_np(v):
    return jnp.pad(v.reshape(N), (0, NP - N))


def kernel(x, edge_index, W1, a_src1, a_dst1, b1, W2, a_src2, a_dst2, b2,
           W3, a_src3, a_dst3, b3, bn_gamma, bn_beta,
           lin1_W, lin1_b, lin2_W, lin2_b):
    E = edge_index.shape[1]
    EAP = _ceil_to(E + N, NW * C)
    padlen = EAP - (E + N)
    edge_a, edge_b = _make_edge_kernels(EAP)

    ei = edge_index.astype(jnp.int32)
    loops = jnp.arange(N, dtype=jnp.int32)
    srcf = jnp.concatenate([ei[0], loops,
                            jnp.zeros((padlen,), jnp.int32)])
    dstf = jnp.concatenate([ei[1], loops,
                            jnp.full((padlen,), N, jnp.int32)])
    src2 = srcf.reshape(-1, 128)
    dst2 = dstf.reshape(-1, 128)

    h, a_s, a_d, mu = _prep1(x, W1, a_src1.reshape(1, H), a_dst1.reshape(1, H))
    for (Wl, asl, adl, bl) in ((W2, a_src2, a_dst2, b1),
                               (W3, a_src3, a_dst3, b2)):
        p_e, dpart = edge_a(src2, dst2, _pad_np(a_s), _pad_np(a_d),
                            _pad_np(mu))
        outp = edge_b(src2, dst2, p_e, h, dpart)
        h, a_s, a_d, mu = _prep23(outp, bl.reshape(1, H), Wl,
                                  asl.reshape(1, H), adl.reshape(1, H))
    p_e, dpart = edge_a(src2, dst2, _pad_np(a_s), _pad_np(a_d), _pad_np(mu))
    outp = edge_b(src2, dst2, p_e, h, dpart)

    w2p = jnp.zeros((H, 128), _f32).at[:, :O].set(lin2_W)
    b2p = jnp.zeros((128,), _f32).at[:O].set(lin2_b)
    out = _head(outp, b3.reshape(1, H), bn_gamma[None, :], bn_beta[None, :],
                lin1_W, lin1_b[None, :], w2p, b2p[None, :])
    return out[:, :O]


# speedup vs baseline: 8.7685x; 1.0248x over previous
import functools

import jax
import jax.numpy as jnp
from jax import lax
from jax.experimental import pallas as pl
from jax.experimental.pallas import tpu as pltpu
from jax.experimental.pallas import tpu_sc as plsc

N = 10000
D = 128
H = 64
O = 3

NP = 10240
NPS = NP // 16
C = 512
RPC = C // 128
NW = 32


def _ceil_to(x, m):
    return (x + m - 1) // m * m


def _edge_a_body(ncH, T, src2, dst2, asrc, adst, mub, p_out, den_out,
                 asrc_v, adst_v, mub_v, sidx2_v, didx2_v, p_v, zb_v,
                 den_sh, lsem, ssem):
    c = lax.axis_index("c")
    s = lax.axis_index("s")
    w = s * 2 + c
    pltpu.sync_copy(asrc, asrc_v)
    pltpu.sync_copy(adst, adst_v)
    pltpu.sync_copy(mub, mub_v)
    for z in range(NPS // 16):
        zb_v[pl.ds(z * 16, 16)] = jnp.zeros((16,), jnp.float32)
    pltpu.sync_copy(zb_v, den_sh.at[pl.ds(s * NPS, NPS)])
    plsc.subcore_barrier()

    def chunk(i, carry):
        base = w * T + i * C
        row0 = w * (T // 128) + i * RPC
        pltpu.sync_copy(src2.at[pl.ds(row0, RPC)], sidx2_v)
        pltpu.sync_copy(dst2.at[pl.ds(row0, RPC)], didx2_v)
        for k in range(C // 16):
            r, kk = divmod(k, 8)
            si = sidx2_v[r, pl.ds(kk * 16, 16)]
            di = didx2_v[r, pl.ds(kk * 16, 16)]
            a1 = plsc.load_gather(asrc_v, [si])
            a2 = plsc.load_gather(adst_v, [di])
            mu = plsc.load_gather(mub_v, [di])
            x = a1 + a2
            e = jnp.maximum(x, 0.2 * x)
            p_v[pl.ds(k * 16, 16)] = jnp.exp(e - mu)
        pltpu.sync_copy(p_v, p_out.at[pl.ds(base, C)])
        descs = [pltpu.async_copy(p_v.at[pl.ds(r * 128, 128)],
                                  den_sh.at[didx2_v.at[r]], ssem, add=True)
                 for r in range(RPC)]
        for dsc in descs:
            dsc.wait()
        return carry

    lax.fori_loop(0, ncH, chunk, 0)
    plsc.subcore_barrier()
    pltpu.sync_copy(den_sh.at[pl.ds(s * NPS, NPS)],
                    den_out.at[c, pl.ds(s * NPS, NPS)])


def _edge_b_body(ncH, T, src2, dst2, p_in, h, dpart, out_part,
                 invd_v, dtmp_v, sidx2_v, didx2_v, p_v, hbuf, zb2_v,
                 out_sh, lsem, gsem, ssem):
    c = lax.axis_index("c")
    s = lax.axis_index("s")
    w = s * 2 + c
    pltpu.sync_copy(dpart.at[0], invd_v)
    pltpu.sync_copy(dpart.at[1], dtmp_v)

    def inv_body(k, carry):
        v = invd_v[pl.ds(k * 16, 16)] + dtmp_v[pl.ds(k * 16, 16)]
        invd_v[pl.ds(k * 16, 16)] = 1.0 / (v + 1e-16)
        return carry

    lax.fori_loop(0, NP // 16, inv_body, 0)

    def zb_body(r, carry):
        for q in range(4):
            zb2_v[r, pl.ds(q * 16, 16)] = jnp.zeros((16,), jnp.float32)
        return carry

    lax.fori_loop(0, 128, zb_body, 0)
    for bloc in range(NPS // 128):
        pltpu.sync_copy(zb2_v, out_sh.at[pl.ds(s * NPS + bloc * 128, 128)])
    plsc.subcore_barrier()

    iota16 = lax.iota(jnp.int32, 16)

    def chunk(i, carry):
        base = w * T + i * C
        row0 = w * (T // 128) + i * RPC
        pltpu.sync_copy(src2.at[pl.ds(row0, RPC)], sidx2_v)
        pltpu.sync_copy(dst2.at[pl.ds(row0, RPC)], didx2_v)
        pltpu.sync_copy(p_in.at[pl.ds(base, C)], p_v)
        gds = [pltpu.async_copy(h.at[sidx2_v.at[r]],
                                hbuf.at[pl.ds(r * 128, 128)], gsem)
               for r in range(RPC)]
        for dsc in gds:
            dsc.wait()

        for r in range(RPC):
            def scale(kk, carry2, r=r):
                di = didx2_v[r, pl.ds(kk * 16, 16)]
                iv = plsc.load_gather(invd_v, [di])
                al = p_v[pl.ds(r * 128 + kk * 16, 16)] * iv
                rows = r * 128 + kk * 16 + iota16
                for j in range(H):
                    cols = jnp.full((16,), j, jnp.int32)
                    v = plsc.load_gather(hbuf, [rows, cols])
                    plsc.store_scatter(hbuf, [rows, cols], v * al)
                return carry2

            lax.fori_loop(0, 8, scale, 0)
        sds = [pltpu.async_copy(hbuf.at[pl.ds(r * 128, 128)],
                                out_sh.at[didx2_v.at[r]], ssem, add=True)
               for r in range(RPC)]
        for dsc in sds:
            dsc.wait()
        return carry

    lax.fori_loop(0, ncH, chunk, 0)
    plsc.subcore_barrier()
    pltpu.sync_copy(out_sh.at[pl.ds(s * NPS, NPS)],
                    out_part.at[c, pl.ds(s * NPS, NPS)])


def _make_edge_kernels(EAP):
    T = EAP // NW
    ncH = T // C
    mesh = plsc.VectorSubcoreMesh(core_axis_name="c", subcore_axis_name="s")
    f32 = jnp.float32
    i32 = jnp.int32
    cparams = pltpu.CompilerParams(needs_layout_passes=False,
                                   use_tc_tiling_on_sc=False)
    edge_a = functools.partial(
        pl.kernel,
        functools.partial(_edge_a_body, ncH, T),
        out_type=(jax.ShapeDtypeStruct((EAP,), f32),
                  jax.ShapeDtypeStruct((2, NP), f32)),
        mesh=mesh,
        scratch_types=[
            pltpu.VMEM((NP,), f32), pltpu.VMEM((NP,), f32),
            pltpu.VMEM((NP,), f32),
            pltpu.VMEM((RPC, 128), i32), pltpu.VMEM((RPC, 128), i32),
            pltpu.VMEM((C,), f32), pltpu.VMEM((NPS,), f32),
            pltpu.VMEM_SHARED((NP,), f32),
            pltpu.SemaphoreType.DMA, pltpu.SemaphoreType.DMA,
        ],
        compiler_params=cparams,
    )()
    edge_b = functools.partial(
        pl.kernel,
        functools.partial(_edge_b_body, ncH, T),
        out_type=jax.ShapeDtypeStruct((2, NP, H), f32),
        mesh=mesh,
        scratch_types=[
            pltpu.VMEM((NP,), f32), pltpu.VMEM((NP,), f32),
            pltpu.VMEM((RPC, 128), i32), pltpu.VMEM((RPC, 128), i32),
            pltpu.VMEM((C,), f32), pltpu.VMEM((C, H), f32),
            pltpu.VMEM((128, H), f32),
            pltpu.VMEM_SHARED((NP, H), f32),
            pltpu.SemaphoreType.DMA, pltpu.SemaphoreType.DMA,
            pltpu.SemaphoreType.DMA,
        ],
        compiler_params=cparams,
    )()
    return edge_a, edge_b


def _prep1_body(x_ref, w_ref, asv_ref, adv_ref, oh, oas, oad, omu):
    h = jnp.dot(x_ref[...], w_ref[...], preferred_element_type=jnp.float32)
    oh[...] = h
    a_s = jnp.sum(h * asv_ref[...], axis=1, keepdims=True)
    a_d = jnp.sum(h * adv_ref[...], axis=1, keepdims=True)
    oas[...] = a_s
    oad[...] = a_d
    z = jnp.max(a_s) + a_d
    omu[...] = jnp.maximum(z, 0.2 * z)


def _prep23_body(part_ref, b_ref, w_ref, asv_ref, adv_ref, oh, oas, oad, omu):
    g = jnp.maximum(part_ref[0, :N, :] + part_ref[1, :N, :] + b_ref[...], 0.0)
    h = jnp.dot(g, w_ref[...], preferred_element_type=jnp.float32)
    oh[...] = h
    a_s = jnp.sum(h * asv_ref[...], axis=1, keepdims=True)
    a_d = jnp.sum(h * adv_ref[...], axis=1, keepdims=True)
    oas[...] = a_s
    oad[...] = a_d
    z = jnp.max(a_s) + a_d
    omu[...] = jnp.maximum(z, 0.2 * z)


def _head_body(part_ref, b_ref, gamma_ref, beta_ref, w1_ref, b1_ref, w2_ref,
               b2_ref, out_ref):
    g = jnp.maximum(part_ref[0, :N, :] + part_ref[1, :N, :] + b_ref[...], 0.0)
    mean = jnp.mean(g, axis=0, keepdims=True)
    var = jnp.mean((g - mean) ** 2, axis=0, keepdims=True)
    hn = (g - mean) * lax.rsqrt(var + 1e-5) * gamma_ref[...] + beta_ref[...]
    z = jnp.maximum(jnp.dot(hn, w1_ref[...], preferred_element_type=jnp.float32)
                    + b1_ref[...], 0.0)
    out_ref[...] = (jnp.dot(z, w2_ref[...], preferred_element_type=jnp.float32)
                    + b2_ref[...])


_f32 = jnp.float32

_prep1 = pl.pallas_call(
    _prep1_body,
    out_shape=(jax.ShapeDtypeStruct((N, H), _f32),
               jax.ShapeDtypeStruct((N, 1), _f32),
               jax.ShapeDtypeStruct((N, 1), _f32),
               jax.ShapeDtypeStruct((N, 1), _f32)))

_prep23 = pl.pallas_call(
    _prep23_body,
    out_shape=(jax.ShapeDtypeStruct((N, H), _f32),
               jax.ShapeDtypeStruct((N, 1), _f32),
               jax.ShapeDtypeStruct((N, 1), _f32),
               jax.ShapeDtypeStruct((N, 1), _f32)))

_head = pl.pallas_call(
    _head_body,
    out_shape=jax.ShapeDtypeStruct((N, 128), _f32))


def _pad_np(v):
    return jnp.pad(v.reshape(N), (0, NP - N))


def kernel(x, edge_index, W1, a_src1, a_dst1, b1, W2, a_src2, a_dst2, b2,
           W3, a_src3, a_dst3, b3, bn_gamma, bn_beta,
           lin1_W, lin1_b, lin2_W, lin2_b):
    E = edge_index.shape[1]
    EAP = _ceil_to(E + N, NW * C)
    padlen = EAP - (E + N)
    edge_a, edge_b = _make_edge_kernels(EAP)

    ei = edge_index.astype(jnp.int32)
    loops = jnp.arange(N, dtype=jnp.int32)
    srcf = jnp.concatenate([ei[0], loops,
                            jnp.zeros((padlen,), jnp.int32)])
    dstf = jnp.concatenate([ei[1], loops,
                            jnp.full((padlen,), N, jnp.int32)])
    src2 = srcf.reshape(-1, 128)
    dst2 = dstf.reshape(-1, 128)

    h, a_s, a_d, mu = _prep1(x, W1, a_src1.reshape(1, H), a_dst1.reshape(1, H))
    for (Wl, asl, adl, bl) in ((W2, a_src2, a_dst2, b1),
                               (W3, a_src3, a_dst3, b2)):
        p_e, dpart = edge_a(src2, dst2, _pad_np(a_s), _pad_np(a_d),
                            _pad_np(mu))
        outp = edge_b(src2, dst2, p_e, h, dpart)
        h, a_s, a_d, mu = _prep23(outp, bl.reshape(1, H), Wl,
                                  asl.reshape(1, H), adl.reshape(1, H))
    p_e, dpart = edge_a(src2, dst2, _pad_np(a_s), _pad_np(a_d), _pad_np(mu))
    outp = edge_b(src2, dst2, p_e, h, dpart)

    w2p = jnp.zeros((H, 128), _f32).at[:, :O].set(lin2_W)
    b2p = jnp.zeros((128,), _f32).at[:O].set(lin2_b)
    out = _head(outp, b3.reshape(1, H), bn_gamma[None, :], bn_beta[None, :],
                lin1_W, lin1_b[None, :], w2p, b2p[None, :])
    return out[:, :O]


# speedup vs baseline: 18.0766x; 2.0615x over previous
import functools

import jax
import jax.numpy as jnp
from jax import lax
from jax.experimental import pallas as pl
from jax.experimental.pallas import tpu as pltpu
from jax.experimental.pallas import tpu_sc as plsc

N = 10000
D = 128
H = 64
O = 3

NP = 10240
NPS = NP // 16
C = 512
RPC = C // 128
NW = 32


def _ceil_to(x, m):
    return (x + m - 1) // m * m


def _edge_a_body(ncH, T, src2, dst2, asrc, adst, mub, p_out, den_out,
                 asrc_v, adst_v, mub_v, sidx2_v, didx2_v, p_v, zb_v,
                 den_sh, lsem, ssem):
    c = lax.axis_index("c")
    s = lax.axis_index("s")
    w = s * 2 + c
    pltpu.sync_copy(asrc, asrc_v)
    pltpu.sync_copy(adst, adst_v)
    pltpu.sync_copy(mub, mub_v)
    for z in range(NPS // 16):
        zb_v[pl.ds(z * 16, 16)] = jnp.zeros((16,), jnp.float32)
    pltpu.sync_copy(zb_v, den_sh.at[pl.ds(s * NPS, NPS)])
    plsc.subcore_barrier()

    def chunk(i, carry):
        base = w * T + i * C
        row0 = w * (T // 128) + i * RPC
        pltpu.sync_copy(src2.at[pl.ds(row0, RPC)], sidx2_v)
        pltpu.sync_copy(dst2.at[pl.ds(row0, RPC)], didx2_v)
        for k in range(C // 16):
            r, kk = divmod(k, 8)
            si = sidx2_v[r, pl.ds(kk * 16, 16)]
            di = didx2_v[r, pl.ds(kk * 16, 16)]
            a1 = plsc.load_gather(asrc_v, [si])
            a2 = plsc.load_gather(adst_v, [di])
            mu = plsc.load_gather(mub_v, [di])
            x = a1 + a2
            e = jnp.maximum(x, 0.2 * x)
            p_v[pl.ds(k * 16, 16)] = jnp.exp(e - mu)
        pltpu.sync_copy(p_v, p_out.at[pl.ds(base, C)])
        descs = [pltpu.async_copy(p_v.at[pl.ds(r * 128, 128)],
                                  den_sh.at[didx2_v.at[r]], ssem, add=True)
                 for r in range(RPC)]
        for dsc in descs:
            dsc.wait()
        return carry

    lax.fori_loop(0, ncH, chunk, 0)
    plsc.subcore_barrier()
    pltpu.sync_copy(den_sh.at[pl.ds(s * NPS, NPS)],
                    den_out.at[c, pl.ds(s * NPS, NPS)])


def _edge_b_body(ncH, T, src2, dst2, p_in, h, dpart, out_part,
                 invd_v, dtmp_v, sidx2_v, didx2_v, p_v, al_v, hbuf, zb2_v,
                 out_sh, lsem, gsem, ssem):
    c = lax.axis_index("c")
    s = lax.axis_index("s")
    w = s * 2 + c
    pltpu.sync_copy(dpart.at[0], invd_v)
    pltpu.sync_copy(dpart.at[1], dtmp_v)

    def inv_body(k, carry):
        v = invd_v[pl.ds(k * 16, 16)] + dtmp_v[pl.ds(k * 16, 16)]
        invd_v[pl.ds(k * 16, 16)] = 1.0 / (v + 1e-16)
        return carry

    lax.fori_loop(0, NP // 16, inv_body, 0)

    def zb_body(r, carry):
        for q in range(4):
            zb2_v[r, pl.ds(q * 16, 16)] = jnp.zeros((16,), jnp.float32)
        return carry

    lax.fori_loop(0, 128, zb_body, 0)
    for bloc in range(NPS // 128):
        pltpu.sync_copy(zb2_v, out_sh.at[pl.ds(s * NPS + bloc * 128, 128)])
    plsc.subcore_barrier()

    iota16 = lax.iota(jnp.int32, 16)

    def chunk(i, carry):
        base = w * T + i * C
        row0 = w * (T // 128) + i * RPC
        pltpu.sync_copy(src2.at[pl.ds(row0, RPC)], sidx2_v)
        pltpu.sync_copy(dst2.at[pl.ds(row0, RPC)], didx2_v)
        pltpu.sync_copy(p_in.at[pl.ds(base, C)], p_v)
        gds = [pltpu.async_copy(h.at[sidx2_v.at[r]],
                                hbuf.at[pl.ds(r * 128, 128)], gsem)
               for r in range(RPC)]
        for k in range(C // 16):
            r, kk = divmod(k, 8)
            di = didx2_v[r, pl.ds(kk * 16, 16)]
            iv = plsc.load_gather(invd_v, [di])
            al_v[pl.ds(k * 16, 16)] = p_v[pl.ds(k * 16, 16)] * iv
        for dsc in gds:
            dsc.wait()

        def scale(g, carry2):
            al16 = al_v[pl.ds(g * 16, 16)]
            e0 = g * 16
            for j in range(16):
                a = al16[j]
                for q in range(H // 16):
                    hbuf[e0 + j, pl.ds(q * 16, 16)] = (
                        hbuf[e0 + j, pl.ds(q * 16, 16)] * a)
            return carry2

        lax.fori_loop(0, C // 16, scale, 0)
        sds = [pltpu.async_copy(hbuf.at[pl.ds(r * 128, 128)],
                                out_sh.at[didx2_v.at[r]], ssem, add=True)
               for r in range(RPC)]
        for dsc in sds:
            dsc.wait()
        return carry

    lax.fori_loop(0, ncH, chunk, 0)
    plsc.subcore_barrier()
    pltpu.sync_copy(out_sh.at[pl.ds(s * NPS, NPS)],
                    out_part.at[c, pl.ds(s * NPS, NPS)])


def _make_edge_kernels(EAP):
    T = EAP // NW
    ncH = T // C
    mesh = plsc.VectorSubcoreMesh(core_axis_name="c", subcore_axis_name="s")
    f32 = jnp.float32
    i32 = jnp.int32
    cparams = pltpu.CompilerParams(needs_layout_passes=False,
                                   use_tc_tiling_on_sc=False)
    edge_a = functools.partial(
        pl.kernel,
        functools.partial(_edge_a_body, ncH, T),
        out_type=(jax.ShapeDtypeStruct((EAP,), f32),
                  jax.ShapeDtypeStruct((2, NP), f32)),
        mesh=mesh,
        scratch_types=[
            pltpu.VMEM((NP,), f32), pltpu.VMEM((NP,), f32),
            pltpu.VMEM((NP,), f32),
            pltpu.VMEM((RPC, 128), i32), pltpu.VMEM((RPC, 128), i32),
            pltpu.VMEM((C,), f32), pltpu.VMEM((NPS,), f32),
            pltpu.VMEM_SHARED((NP,), f32),
            pltpu.SemaphoreType.DMA, pltpu.SemaphoreType.DMA,
        ],
        compiler_params=cparams,
    )()
    edge_b = functools.partial(
        pl.kernel,
        functools.partial(_edge_b_body, ncH, T),
        out_type=jax.ShapeDtypeStruct((2, NP, H), f32),
        mesh=mesh,
        scratch_types=[
            pltpu.VMEM((NP,), f32), pltpu.VMEM((NP,), f32),
            pltpu.VMEM((RPC, 128), i32), pltpu.VMEM((RPC, 128), i32),
            pltpu.VMEM((C,), f32), pltpu.VMEM((C,), f32),
            pltpu.VMEM((C, H), f32),
            pltpu.VMEM((128, H), f32),
            pltpu.VMEM_SHARED((NP, H), f32),
            pltpu.SemaphoreType.DMA, pltpu.SemaphoreType.DMA,
            pltpu.SemaphoreType.DMA,
        ],
        compiler_params=cparams,
    )()
    return edge_a, edge_b


def _prep1_body(x_ref, w_ref, asv_ref, adv_ref, oh, oas, oad, omu):
    h = jnp.dot(x_ref[...], w_ref[...], preferred_element_type=jnp.float32)
    oh[...] = h
    a_s = jnp.sum(h * asv_ref[...], axis=1, keepdims=True)
    a_d = jnp.sum(h * adv_ref[...], axis=1, keepdims=True)
    oas[...] = a_s
    oad[...] = a_d
    z = jnp.max(a_s) + a_d
    omu[...] = jnp.maximum(z, 0.2 * z)


def _prep23_body(part_ref, b_ref, w_ref, asv_ref, adv_ref, oh, oas, oad, omu):
    g = jnp.maximum(part_ref[0, :N, :] + part_ref[1, :N, :] + b_ref[...], 0.0)
    h = jnp.dot(g, w_ref[...], preferred_element_type=jnp.float32)
    oh[...] = h
    a_s = jnp.sum(h * asv_ref[...], axis=1, keepdims=True)
    a_d = jnp.sum(h * adv_ref[...], axis=1, keepdims=True)
    oas[...] = a_s
    oad[...] = a_d
    z = jnp.max(a_s) + a_d
    omu[...] = jnp.maximum(z, 0.2 * z)


def _head_body(part_ref, b_ref, gamma_ref, beta_ref, w1_ref, b1_ref, w2_ref,
               b2_ref, out_ref):
    g = jnp.maximum(part_ref[0, :N, :] + part_ref[1, :N, :] + b_ref[...], 0.0)
    mean = jnp.mean(g, axis=0, keepdims=True)
    var = jnp.mean((g - mean) ** 2, axis=0, keepdims=True)
    hn = (g - mean) * lax.rsqrt(var + 1e-5) * gamma_ref[...] + beta_ref[...]
    z = jnp.maximum(jnp.dot(hn, w1_ref[...], preferred_element_type=jnp.float32)
                    + b1_ref[...], 0.0)
    out_ref[...] = (jnp.dot(z, w2_ref[...], preferred_element_type=jnp.float32)
                    + b2_ref[...])


_f32 = jnp.float32

_prep1 = pl.pallas_call(
    _prep1_body,
    out_shape=(jax.ShapeDtypeStruct((N, H), _f32),
               jax.ShapeDtypeStruct((N, 1), _f32),
               jax.ShapeDtypeStruct((N, 1), _f32),
               jax.ShapeDtypeStruct((N, 1), _f32)))

_prep23 = pl.pallas_call(
    _prep23_body,
    out_shape=(jax.ShapeDtypeStruct((N, H), _f32),
               jax.ShapeDtypeStruct((N, 1), _f32),
               jax.ShapeDtypeStruct((N, 1), _f32),
               jax.ShapeDtypeStruct((N, 1), _f32)))

_head = pl.pallas_call(
    _head_body,
    out_shape=jax.ShapeDtypeStruct((N, 128), _f32))


def _pad_np(v):
    return jnp.pad(v.reshape(N), (0, NP - N))


def kernel(x, edge_index, W1, a_src1, a_dst1, b1, W2, a_src2, a_dst2, b2,
           W3, a_src3, a_dst3, b3, bn_gamma, bn_beta,
           lin1_W, lin1_b, lin2_W, lin2_b):
    E = edge_index.shape[1]
    EAP = _ceil_to(E + N, NW * C)
    padlen = EAP - (E + N)
    edge_a, edge_b = _make_edge_kernels(EAP)

    ei = edge_index.astype(jnp.int32)
    loops = jnp.arange(N, dtype=jnp.int32)
    srcf = jnp.concatenate([ei[0], loops,
                            jnp.zeros((padlen,), jnp.int32)])
    dstf = jnp.concatenate([ei[1], loops,
                            jnp.full((padlen,), N, jnp.int32)])
    src2 = srcf.reshape(-1, 128)
    dst2 = dstf.reshape(-1, 128)

    h, a_s, a_d, mu = _prep1(x, W1, a_src1.reshape(1, H), a_dst1.reshape(1, H))
    for (Wl, asl, adl, bl) in ((W2, a_src2, a_dst2, b1),
                               (W3, a_src3, a_dst3, b2)):
        p_e, dpart = edge_a(src2, dst2, _pad_np(a_s), _pad_np(a_d),
                            _pad_np(mu))
        outp = edge_b(src2, dst2, p_e, h, dpart)
        h, a_s, a_d, mu = _prep23(outp, bl.reshape(1, H), Wl,
                                  asl.reshape(1, H), adl.reshape(1, H))
    p_e, dpart = edge_a(src2, dst2, _pad_np(a_s), _pad_np(a_d), _pad_np(mu))
    outp = edge_b(src2, dst2, p_e, h, dpart)

    w2p = jnp.zeros((H, 128), _f32).at[:, :O].set(lin2_W)
    b2p = jnp.zeros((128,), _f32).at[:O].set(lin2_b)
    out = _head(outp, b3.reshape(1, H), bn_gamma[None, :], bn_beta[None, :],
                lin1_W, lin1_b[None, :], w2p, b2p[None, :])
    return out[:, :O]


# speedup vs baseline: 23.1266x; 1.2794x over previous
import functools

import jax
import jax.numpy as jnp
from jax import lax
from jax.experimental import pallas as pl
from jax.experimental.pallas import tpu as pltpu
from jax.experimental.pallas import tpu_sc as plsc

N = 10000
D = 128
H = 64
O = 3

NP = 10240
NPS = NP // 16
C = 256
RPC = C // 128
NW = 32


def _ceil_to(x, m):
    return (x + m - 1) // m * m


def _edge_a_body(ncH, T, src2, dst2, asrc, adst, mub, p_out, den_out,
                 asrc_v, adst_v, mub_v, sidx2_v, didx2_v, p_v, zb_v,
                 den_sh, lsem, ssem):
    c = lax.axis_index("c")
    s = lax.axis_index("s")
    w = s * 2 + c
    pltpu.sync_copy(asrc, asrc_v)
    pltpu.sync_copy(adst, adst_v)
    pltpu.sync_copy(mub, mub_v)
    for z in range(NPS // 16):
        zb_v[pl.ds(z * 16, 16)] = jnp.zeros((16,), jnp.float32)
    pltpu.sync_copy(zb_v, den_sh.at[pl.ds(s * NPS, NPS)])
    plsc.subcore_barrier()

    def chunk(i, carry):
        base = w * T + i * C
        row0 = w * (T // 128) + i * RPC
        pltpu.sync_copy(src2.at[pl.ds(row0, RPC)], sidx2_v)
        pltpu.sync_copy(dst2.at[pl.ds(row0, RPC)], didx2_v)
        for k in range(C // 16):
            r, kk = divmod(k, 8)
            si = sidx2_v[r, pl.ds(kk * 16, 16)]
            di = didx2_v[r, pl.ds(kk * 16, 16)]
            a1 = plsc.load_gather(asrc_v, [si])
            a2 = plsc.load_gather(adst_v, [di])
            mu = plsc.load_gather(mub_v, [di])
            x = a1 + a2
            e = jnp.maximum(x, 0.2 * x)
            p_v[pl.ds(k * 16, 16)] = jnp.exp(e - mu)
        pltpu.sync_copy(p_v, p_out.at[pl.ds(base, C)])
        descs = [pltpu.async_copy(p_v.at[pl.ds(r * 128, 128)],
                                  den_sh.at[didx2_v.at[r]], ssem, add=True)
                 for r in range(RPC)]
        for dsc in descs:
            dsc.wait()
        return carry

    lax.fori_loop(0, ncH, chunk, 0)
    plsc.subcore_barrier()
    pltpu.sync_copy(den_sh.at[pl.ds(s * NPS, NPS)],
                    den_out.at[c, pl.ds(s * NPS, NPS)])


def _edge_b_body(ncH, T, src2, dst2, p_in, h, dpart, out_part,
                 invd_v, dtmp_v,
                 sidx0, sidx1, sidx2, didx0, didx1, didx2,
                 p0, p1, p2, al0, al1, al2, hb0, hb1, hb2, zb2_v,
                 out_sh, gs0, gs1, gs2, ss0, ss1, ss2):
    c = lax.axis_index("c")
    s = lax.axis_index("s")
    w = s * 2 + c
    sidx = [sidx0, sidx1, sidx2]
    didx = [didx0, didx1, didx2]
    pb = [p0, p1, p2]
    alb = [al0, al1, al2]
    hb = [hb0, hb1, hb2]
    gs = [gs0, gs1, gs2]
    ss = [ss0, ss1, ss2]
    pltpu.sync_copy(dpart.at[0], invd_v)
    pltpu.sync_copy(dpart.at[1], dtmp_v)

    def inv_body(k, carry):
        v = invd_v[pl.ds(k * 16, 16)] + dtmp_v[pl.ds(k * 16, 16)]
        invd_v[pl.ds(k * 16, 16)] = 1.0 / (v + 1e-16)
        return carry

    lax.fori_loop(0, NP // 16, inv_body, 0)

    def zb_body(r, carry):
        for q in range(4):
            zb2_v[r, pl.ds(q * 16, 16)] = jnp.zeros((16,), jnp.float32)
        return carry

    lax.fori_loop(0, 64, zb_body, 0)
    for bloc in range(NPS // 64):
        pltpu.sync_copy(zb2_v, out_sh.at[pl.ds(s * NPS + bloc * 64, 64)])
    plsc.subcore_barrier()

    def stage(slot, j):
        base = w * T + j * C
        row0 = w * (T // 128) + j * RPC
        pltpu.sync_copy(src2.at[pl.ds(row0, RPC)], sidx[slot])
        pltpu.sync_copy(dst2.at[pl.ds(row0, RPC)], didx[slot])
        pltpu.sync_copy(p_in.at[pl.ds(base, C)], pb[slot])
        for r in range(RPC):
            pltpu.async_copy(h.at[sidx[slot].at[r]],
                             hb[slot].at[pl.ds(r * 128, 128)], gs[slot])

    def drain_gather(slot):
        for r in range(RPC):
            pltpu.make_async_copy(h.at[sidx[slot].at[r]],
                                  hb[slot].at[pl.ds(r * 128, 128)],
                                  gs[slot]).wait()

    def issue_scatter(slot):
        for r in range(RPC):
            pltpu.async_copy(hb[slot].at[pl.ds(r * 128, 128)],
                             out_sh.at[didx[slot].at[r]], ss[slot], add=True)

    def drain_scatter(slot):
        for r in range(RPC):
            pltpu.make_async_copy(hb[slot].at[pl.ds(r * 128, 128)],
                                  out_sh.at[didx[slot].at[r]],
                                  ss[slot]).wait()

    nfill = jnp.full((16,), N, jnp.int32)
    for slot in (1, 2):
        for r in range(RPC):
            for q in range(8):
                didx[slot][r, pl.ds(q * 16, 16)] = nfill
        issue_scatter(slot)
    stage(0, 0)

    def outer(g, carry):
        for b in range(3):
            j = g * 3 + b
            sn = (b + 1) % 3
            drain_scatter(sn)
            stage(sn, jnp.minimum(j + 1, ncH - 1))
            drain_gather(b)
            for k in range(C // 16):
                r, kk = divmod(k, 8)
                di = didx[b][r, pl.ds(kk * 16, 16)]
                iv = plsc.load_gather(invd_v, [di])
                alb[b][pl.ds(k * 16, 16)] = pb[b][pl.ds(k * 16, 16)] * iv

            def scale(g2, carry2, b=b):
                al16 = alb[b][pl.ds(g2 * 16, 16)]
                e0 = g2 * 16
                for jl in range(16):
                    a = al16[jl]
                    for q in range(H // 16):
                        hb[b][e0 + jl, pl.ds(q * 16, 16)] = (
                            hb[b][e0 + jl, pl.ds(q * 16, 16)] * a)
                return carry2

            lax.fori_loop(0, C // 16, scale, 0)
            issue_scatter(b)
        return carry

    lax.fori_loop(0, ncH // 3, outer, 0)
    drain_scatter(1)
    drain_scatter(2)
    drain_gather(0)
    plsc.subcore_barrier()
    pltpu.sync_copy(out_sh.at[pl.ds(s * NPS, NPS)],
                    out_part.at[c, pl.ds(s * NPS, NPS)])


def _make_edge_kernels(EAP):
    T = EAP // NW
    ncH = T // C
    mesh = plsc.VectorSubcoreMesh(core_axis_name="c", subcore_axis_name="s")
    f32 = jnp.float32
    i32 = jnp.int32
    cparams = pltpu.CompilerParams(needs_layout_passes=False,
                                   use_tc_tiling_on_sc=False)
    edge_a = functools.partial(
        pl.kernel,
        functools.partial(_edge_a_body, ncH, T),
        out_type=(jax.ShapeDtypeStruct((EAP,), f32),
                  jax.ShapeDtypeStruct((2, NP), f32)),
        mesh=mesh,
        scratch_types=[
            pltpu.VMEM((NP,), f32), pltpu.VMEM((NP,), f32),
            pltpu.VMEM((NP,), f32),
            pltpu.VMEM((RPC, 128), i32), pltpu.VMEM((RPC, 128), i32),
            pltpu.VMEM((C,), f32), pltpu.VMEM((NPS,), f32),
            pltpu.VMEM_SHARED((NP,), f32),
            pltpu.SemaphoreType.DMA, pltpu.SemaphoreType.DMA,
        ],
        compiler_params=cparams,
    )()
    edge_b = functools.partial(
        pl.kernel,
        functools.partial(_edge_b_body, ncH, T),
        out_type=jax.ShapeDtypeStruct((2, NP, H), f32),
        mesh=mesh,
        scratch_types=(
            [pltpu.VMEM((NP,), f32), pltpu.VMEM((NP,), f32)]
            + [pltpu.VMEM((RPC, 128), i32)] * 6
            + [pltpu.VMEM((C,), f32)] * 6
            + [pltpu.VMEM((C, H), f32)] * 3
            + [pltpu.VMEM((64, H), f32),
               pltpu.VMEM_SHARED((NP, H), f32)]
            + [pltpu.SemaphoreType.DMA] * 6
        ),
        compiler_params=cparams,
    )()
    return edge_a, edge_b


def _prep1_body(x_ref, w_ref, asv_ref, adv_ref, oh, oas, oad, omu):
    h = jnp.dot(x_ref[...], w_ref[...], preferred_element_type=jnp.float32)
    oh[...] = h
    a_s = jnp.sum(h * asv_ref[...], axis=1, keepdims=True)
    a_d = jnp.sum(h * adv_ref[...], axis=1, keepdims=True)
    oas[...] = a_s
    oad[...] = a_d
    z = jnp.max(a_s) + a_d
    omu[...] = jnp.maximum(z, 0.2 * z)


def _prep23_body(part_ref, b_ref, w_ref, asv_ref, adv_ref, oh, oas, oad, omu):
    g = jnp.maximum(part_ref[0, :N, :] + part_ref[1, :N, :] + b_ref[...], 0.0)
    h = jnp.dot(g, w_ref[...], preferred_element_type=jnp.float32)
    oh[...] = h
    a_s = jnp.sum(h * asv_ref[...], axis=1, keepdims=True)
    a_d = jnp.sum(h * adv_ref[...], axis=1, keepdims=True)
    oas[...] = a_s
    oad[...] = a_d
    z = jnp.max(a_s) + a_d
    omu[...] = jnp.maximum(z, 0.2 * z)


def _head_body(part_ref, b_ref, gamma_ref, beta_ref, w1_ref, b1_ref, w2_ref,
               b2_ref, out_ref):
    g = jnp.maximum(part_ref[0, :N, :] + part_ref[1, :N, :] + b_ref[...], 0.0)
    mean = jnp.mean(g, axis=0, keepdims=True)
    var = jnp.mean((g - mean) ** 2, axis=0, keepdims=True)
    hn = (g - mean) * lax.rsqrt(var + 1e-5) * gamma_ref[...] + beta_ref[...]
    z = jnp.maximum(jnp.dot(hn, w1_ref[...], preferred_element_type=jnp.float32)
                    + b1_ref[...], 0.0)
    out_ref[...] = (jnp.dot(z, w2_ref[...], preferred_element_type=jnp.float32)
                    + b2_ref[...])


_f32 = jnp.float32

_prep1 = pl.pallas_call(
    _prep1_body,
    out_shape=(jax.ShapeDtypeStruct((N, H), _f32),
               jax.ShapeDtypeStruct((N, 1), _f32),
               jax.ShapeDtypeStruct((N, 1), _f32),
               jax.ShapeDtypeStruct((N, 1), _f32)))

_prep23 = pl.pallas_call(
    _prep23_body,
    out_shape=(jax.ShapeDtypeStruct((N, H), _f32),
               jax.ShapeDtypeStruct((N, 1), _f32),
               jax.ShapeDtypeStruct((N, 1), _f32),
               jax.ShapeDtypeStruct((N, 1), _f32)))

_head = pl.pallas_call(
    _head_body,
    out_shape=jax.ShapeDtypeStruct((N, 128), _f32))


def _pad_np(v):
    return jnp.pad(v.reshape(N), (0, NP - N))


def kernel(x, edge_index, W1, a_src1, a_dst1, b1, W2, a_src2, a_dst2, b2,
           W3, a_src3, a_dst3, b3, bn_gamma, bn_beta,
           lin1_W, lin1_b, lin2_W, lin2_b):
    E = edge_index.shape[1]
    EAP = _ceil_to(E + N, NW * C * 3)
    padlen = EAP - (E + N)
    edge_a, edge_b = _make_edge_kernels(EAP)

    ei = edge_index.astype(jnp.int32)
    loops = jnp.arange(N, dtype=jnp.int32)
    srcf = jnp.concatenate([ei[0], loops,
                            jnp.zeros((padlen,), jnp.int32)])
    dstf = jnp.concatenate([ei[1], loops,
                            jnp.full((padlen,), N, jnp.int32)])
    src2 = srcf.reshape(-1, 128)
    dst2 = dstf.reshape(-1, 128)

    h, a_s, a_d, mu = _prep1(x, W1, a_src1.reshape(1, H), a_dst1.reshape(1, H))
    for (Wl, asl, adl, bl) in ((W2, a_src2, a_dst2, b1),
                               (W3, a_src3, a_dst3, b2)):
        p_e, dpart = edge_a(src2, dst2, _pad_np(a_s), _pad_np(a_d),
                            _pad_np(mu))
        outp = edge_b(src2, dst2, p_e, h, dpart)
        h, a_s, a_d, mu = _prep23(outp, bl.reshape(1, H), Wl,
                                  asl.reshape(1, H), adl.reshape(1, H))
    p_e, dpart = edge_a(src2, dst2, _pad_np(a_s), _pad_np(a_d), _pad_np(mu))
    outp = edge_b(src2, dst2, p_e, h, dpart)

    w2p = jnp.zeros((H, 128), _f32).at[:, :O].set(lin2_W)
    b2p = jnp.zeros((128,), _f32).at[:O].set(lin2_b)
    out = _head(outp, b3.reshape(1, H), bn_gamma[None, :], bn_beta[None, :],
                lin1_W, lin1_b[None, :], w2p, b2p[None, :])
    return out[:, :O]


# speedup vs baseline: 24.2604x; 1.0490x over previous
import functools

import jax
import jax.numpy as jnp
from jax import lax
from jax.experimental import pallas as pl
from jax.experimental.pallas import tpu as pltpu
from jax.experimental.pallas import tpu_sc as plsc

N = 10000
D = 128
H = 64
O = 3

NP = 10240
NPS = NP // 16
C = 256
RPC = C // 128
NW = 32


def _ceil_to(x, m):
    return (x + m - 1) // m * m


def _edge_a_body(ncH, T, CA, src2, dst2, asrc, adst, mub, p_out, den_out,
                 asrc_v, adst_v, mub_v, sidx2_v, didx2_v, p_v, zb_v,
                 den_sh, lsem, ssem):
    RPCA = CA // 128
    c = lax.axis_index("c")
    s = lax.axis_index("s")
    w = s * 2 + c
    pltpu.sync_copy(asrc, asrc_v)
    pltpu.sync_copy(adst, adst_v)
    pltpu.sync_copy(mub, mub_v)
    for z in range(NPS // 16):
        zb_v[pl.ds(z * 16, 16)] = jnp.zeros((16,), jnp.float32)
    pltpu.sync_copy(zb_v, den_sh.at[pl.ds(s * NPS, NPS)])
    plsc.subcore_barrier()

    def chunk(i, carry):
        base = w * T + i * CA
        row0 = w * (T // 128) + i * RPCA
        pltpu.sync_copy(src2.at[pl.ds(row0, RPCA)], sidx2_v)
        pltpu.sync_copy(dst2.at[pl.ds(row0, RPCA)], didx2_v)
        for k in range(CA // 16):
            r, kk = divmod(k, 8)
            si = sidx2_v[r, pl.ds(kk * 16, 16)]
            di = didx2_v[r, pl.ds(kk * 16, 16)]
            a1 = plsc.load_gather(asrc_v, [si])
            a2 = plsc.load_gather(adst_v, [di])
            mu = plsc.load_gather(mub_v, [di])
            x = a1 + a2
            e = jnp.maximum(x, 0.2 * x)
            p_v[pl.ds(k * 16, 16)] = jnp.exp(e - mu)
        pltpu.sync_copy(p_v, p_out.at[pl.ds(base, CA)])
        descs = [pltpu.async_copy(p_v.at[pl.ds(r * 128, 128)],
                                  den_sh.at[didx2_v.at[r]], ssem, add=True)
                 for r in range(RPCA)]
        for dsc in descs:
            dsc.wait()
        return carry

    lax.fori_loop(0, ncH, chunk, 0)
    plsc.subcore_barrier()
    pltpu.sync_copy(den_sh.at[pl.ds(s * NPS, NPS)],
                    den_out.at[c, pl.ds(s * NPS, NPS)])


def _edge_b_body(ncH, T, src2, dst2, p_in, h, dpart, out_part,
                 invd_v, dtmp_v,
                 sidx0, sidx1, sidx2, didx0, didx1, didx2,
                 p0, p1, p2, al0, al1, al2, hb0, hb1, hb2, zb2_v,
                 out_sh, gs0, gs1, gs2, ss0, ss1, ss2,
                 ps0, ps1, ps2, ds0, ds1, ds2):
    c = lax.axis_index("c")
    s = lax.axis_index("s")
    w = s * 2 + c
    sidx = [sidx0, sidx1, sidx2]
    didx = [didx0, didx1, didx2]
    pb = [p0, p1, p2]
    alb = [al0, al1, al2]
    hb = [hb0, hb1, hb2]
    gs = [gs0, gs1, gs2]
    ss = [ss0, ss1, ss2]
    psem = [ps0, ps1, ps2]
    dsem = [ds0, ds1, ds2]
    pltpu.sync_copy(dpart.at[0], invd_v)
    pltpu.sync_copy(dpart.at[1], dtmp_v)

    def inv_body(k, carry):
        v = invd_v[pl.ds(k * 16, 16)] + dtmp_v[pl.ds(k * 16, 16)]
        invd_v[pl.ds(k * 16, 16)] = 1.0 / (v + 1e-16)
        return carry

    lax.fori_loop(0, NP // 16, inv_body, 0)

    def zb_body(r, carry):
        for q in range(4):
            zb2_v[r, pl.ds(q * 16, 16)] = jnp.zeros((16,), jnp.float32)
        return carry

    lax.fori_loop(0, 64, zb_body, 0)
    for bloc in range(NPS // 64):
        pltpu.sync_copy(zb2_v, out_sh.at[pl.ds(s * NPS + bloc * 64, 64)])
    plsc.subcore_barrier()

    def stage(slot, j):
        base = w * T + j * C
        row0 = w * (T // 128) + j * RPC
        pltpu.sync_copy(src2.at[pl.ds(row0, RPC)], sidx[slot])
        pltpu.async_copy(dst2.at[pl.ds(row0, RPC)], didx[slot], dsem[slot])
        pltpu.async_copy(p_in.at[pl.ds(base, C)], pb[slot], psem[slot])
        for r in range(RPC):
            pltpu.async_copy(h.at[sidx[slot].at[r]],
                             hb[slot].at[pl.ds(r * 128, 128)], gs[slot])

    def drain_stage(slot, j):
        base = w * T + j * C
        row0 = w * (T // 128) + j * RPC
        pltpu.make_async_copy(dst2.at[pl.ds(row0, RPC)], didx[slot],
                              dsem[slot]).wait()
        pltpu.make_async_copy(p_in.at[pl.ds(base, C)], pb[slot],
                              psem[slot]).wait()

    def drain_gather(slot):
        for r in range(RPC):
            pltpu.make_async_copy(h.at[sidx[slot].at[r]],
                                  hb[slot].at[pl.ds(r * 128, 128)],
                                  gs[slot]).wait()

    def issue_scatter(slot):
        for r in range(RPC):
            pltpu.async_copy(hb[slot].at[pl.ds(r * 128, 128)],
                             out_sh.at[didx[slot].at[r]], ss[slot], add=True)

    def drain_scatter(slot):
        for r in range(RPC):
            pltpu.make_async_copy(hb[slot].at[pl.ds(r * 128, 128)],
                                  out_sh.at[didx[slot].at[r]],
                                  ss[slot]).wait()

    nfill = jnp.full((16,), N, jnp.int32)
    for slot in (1, 2):
        for r in range(RPC):
            for q in range(8):
                didx[slot][r, pl.ds(q * 16, 16)] = nfill
        issue_scatter(slot)
    stage(0, 0)

    def outer(g, carry):
        for b in range(3):
            j = g * 3 + b
            sn = (b + 1) % 3
            drain_scatter(sn)
            stage(sn, jnp.minimum(j + 1, ncH - 1))
            drain_gather(b)
            drain_stage(b, j)
            for k in range(C // 16):
                r, kk = divmod(k, 8)
                di = didx[b][r, pl.ds(kk * 16, 16)]
                iv = plsc.load_gather(invd_v, [di])
                alb[b][pl.ds(k * 16, 16)] = pb[b][pl.ds(k * 16, 16)] * iv

            def scale(g2, carry2, b=b):
                al16 = alb[b][pl.ds(g2 * 16, 16)]
                e0 = g2 * 16
                for jl in range(16):
                    a = al16[jl]
                    for q in range(H // 16):
                        hb[b][e0 + jl, pl.ds(q * 16, 16)] = (
                            hb[b][e0 + jl, pl.ds(q * 16, 16)] * a)
                return carry2

            lax.fori_loop(0, C // 16, scale, 0)
            issue_scatter(b)
        return carry

    lax.fori_loop(0, ncH // 3, outer, 0)
    drain_scatter(1)
    drain_scatter(2)
    drain_gather(0)
    drain_stage(0, ncH - 1)
    plsc.subcore_barrier()
    pltpu.sync_copy(out_sh.at[pl.ds(s * NPS, NPS)],
                    out_part.at[c, pl.ds(s * NPS, NPS)])


def _make_edge_kernels(EAP):
    T = EAP // NW
    CA = 512
    ncHa = T // CA
    ncH = T // C
    mesh = plsc.VectorSubcoreMesh(core_axis_name="c", subcore_axis_name="s")
    f32 = jnp.float32
    i32 = jnp.int32
    cparams = pltpu.CompilerParams(needs_layout_passes=False,
                                   use_tc_tiling_on_sc=False)
    edge_a = functools.partial(
        pl.kernel,
        functools.partial(_edge_a_body, ncHa, T, CA),
        out_type=(jax.ShapeDtypeStruct((EAP,), f32),
                  jax.ShapeDtypeStruct((2, NP), f32)),
        mesh=mesh,
        scratch_types=[
            pltpu.VMEM((NP,), f32), pltpu.VMEM((NP,), f32),
            pltpu.VMEM((NP,), f32),
            pltpu.VMEM((CA // 128, 128), i32), pltpu.VMEM((CA // 128, 128), i32),
            pltpu.VMEM((CA,), f32), pltpu.VMEM((NPS,), f32),
            pltpu.VMEM_SHARED((NP,), f32),
            pltpu.SemaphoreType.DMA, pltpu.SemaphoreType.DMA,
        ],
        compiler_params=cparams,
    )()
    edge_b = functools.partial(
        pl.kernel,
        functools.partial(_edge_b_body, ncH, T),
        out_type=jax.ShapeDtypeStruct((2, NP, H), f32),
        mesh=mesh,
        scratch_types=(
            [pltpu.VMEM((NP,), f32), pltpu.VMEM((NP,), f32)]
            + [pltpu.VMEM((RPC, 128), i32)] * 6
            + [pltpu.VMEM((C,), f32)] * 6
            + [pltpu.VMEM((C, H), f32)] * 3
            + [pltpu.VMEM((64, H), f32),
               pltpu.VMEM_SHARED((NP, H), f32)]
            + [pltpu.SemaphoreType.DMA] * 12
        ),
        compiler_params=cparams,
    )()
    return edge_a, edge_b


def _prep1_body(x_ref, w_ref, asv_ref, adv_ref, oh, oas, oad, omu):
    h = jnp.dot(x_ref[...], w_ref[...], preferred_element_type=jnp.float32)
    oh[...] = h
    a_s = jnp.sum(h * asv_ref[...], axis=1, keepdims=True)
    a_d = jnp.sum(h * adv_ref[...], axis=1, keepdims=True)
    oas[...] = a_s
    oad[...] = a_d
    z = jnp.max(a_s) + a_d
    omu[...] = jnp.maximum(z, 0.2 * z)


def _prep23_body(part_ref, b_ref, w_ref, asv_ref, adv_ref, oh, oas, oad, omu):
    g = jnp.maximum(part_ref[0, :N, :] + part_ref[1, :N, :] + b_ref[...], 0.0)
    h = jnp.dot(g, w_ref[...], preferred_element_type=jnp.float32)
    oh[...] = h
    a_s = jnp.sum(h * asv_ref[...], axis=1, keepdims=True)
    a_d = jnp.sum(h * adv_ref[...], axis=1, keepdims=True)
    oas[...] = a_s
    oad[...] = a_d
    z = jnp.max(a_s) + a_d
    omu[...] = jnp.maximum(z, 0.2 * z)


def _head_body(part_ref, b_ref, gamma_ref, beta_ref, w1_ref, b1_ref, w2_ref,
               b2_ref, out_ref):
    g = jnp.maximum(part_ref[0, :N, :] + part_ref[1, :N, :] + b_ref[...], 0.0)
    mean = jnp.mean(g, axis=0, keepdims=True)
    var = jnp.mean((g - mean) ** 2, axis=0, keepdims=True)
    hn = (g - mean) * lax.rsqrt(var + 1e-5) * gamma_ref[...] + beta_ref[...]
    z = jnp.maximum(jnp.dot(hn, w1_ref[...], preferred_element_type=jnp.float32)
                    + b1_ref[...], 0.0)
    out_ref[...] = (jnp.dot(z, w2_ref[...], preferred_element_type=jnp.float32)
                    + b2_ref[...])


_f32 = jnp.float32

_prep1 = pl.pallas_call(
    _prep1_body,
    out_shape=(jax.ShapeDtypeStruct((N, H), _f32),
               jax.ShapeDtypeStruct((N, 1), _f32),
               jax.ShapeDtypeStruct((N, 1), _f32),
               jax.ShapeDtypeStruct((N, 1), _f32)))

_prep23 = pl.pallas_call(
    _prep23_body,
    out_shape=(jax.ShapeDtypeStruct((N, H), _f32),
               jax.ShapeDtypeStruct((N, 1), _f32),
               jax.ShapeDtypeStruct((N, 1), _f32),
               jax.ShapeDtypeStruct((N, 1), _f32)))

_head = pl.pallas_call(
    _head_body,
    out_shape=jax.ShapeDtypeStruct((N, 128), _f32))


def _pad_np(v):
    return jnp.pad(v.reshape(N), (0, NP - N))


def kernel(x, edge_index, W1, a_src1, a_dst1, b1, W2, a_src2, a_dst2, b2,
           W3, a_src3, a_dst3, b3, bn_gamma, bn_beta,
           lin1_W, lin1_b, lin2_W, lin2_b):
    E = edge_index.shape[1]
    EAP = _ceil_to(E + N, NW * 1536)
    padlen = EAP - (E + N)
    edge_a, edge_b = _make_edge_kernels(EAP)

    ei = edge_index.astype(jnp.int32)
    loops = jnp.arange(N, dtype=jnp.int32)
    srcf = jnp.concatenate([ei[0], loops,
                            jnp.zeros((padlen,), jnp.int32)])
    dstf = jnp.concatenate([ei[1], loops,
                            jnp.full((padlen,), N, jnp.int32)])
    src2 = srcf.reshape(-1, 128)
    dst2 = dstf.reshape(-1, 128)

    h, a_s, a_d, mu = _prep1(x, W1, a_src1.reshape(1, H), a_dst1.reshape(1, H))
    for (Wl, asl, adl, bl) in ((W2, a_src2, a_dst2, b1),
                               (W3, a_src3, a_dst3, b2)):
        p_e, dpart = edge_a(src2, dst2, _pad_np(a_s), _pad_np(a_d),
                            _pad_np(mu))
        outp = edge_b(src2, dst2, p_e, h, dpart)
        h, a_s, a_d, mu = _prep23(outp, bl.reshape(1, H), Wl,
                                  asl.reshape(1, H), adl.reshape(1, H))
    p_e, dpart = edge_a(src2, dst2, _pad_np(a_s), _pad_np(a_d), _pad_np(mu))
    outp = edge_b(src2, dst2, p_e, h, dpart)

    w2p = jnp.zeros((H, 128), _f32).at[:, :O].set(lin2_W)
    b2p = jnp.zeros((128,), _f32).at[:O].set(lin2_b)
    out = _head(outp, b3.reshape(1, H), bn_gamma[None, :], bn_beta[None, :],
                lin1_W, lin1_b[None, :], w2p, b2p[None, :])
    return out[:, :O]
